# baseline (speedup 1.0000x reference)
import jax
import jax.numpy as jnp
from jax import lax
from jax.experimental import pallas as pl
from jax.experimental.pallas import tpu as pltpu

OUT_D = 1000
B, L = 4096, 20
B0 = 128
NBLK = B // B0


def _body(x_ref, o_ref):
    xb = x_ref[...]
    iota = lax.broadcasted_iota(jnp.int32, (B0, L, OUT_D), 2)
    o_ref[...] = (iota == xb[:, :, None]).astype(jnp.float32)


def kernel(x):
    return pl.pallas_call(
        _body,
        grid=(NBLK,),
        in_specs=[pl.BlockSpec((B0, L), lambda i: (i, 0))],
        out_specs=pl.BlockSpec((B0, L, OUT_D), lambda i: (i, 0, 0)),
        out_shape=jax.ShapeDtypeStruct((B, L, OUT_D), jnp.float32),
        compiler_params=pltpu.CompilerParams(
            dimension_semantics=("parallel",),
        ),
    )(x)
